# baseline (speedup 1.0000x reference)
import functools

import jax
import jax.numpy as jnp
from jax import lax
from jax.experimental import pallas as pl
from jax.experimental.pallas import tpu as pltpu
from jax.experimental.pallas import tpu_sc as plsc

NC = 2
NS = 16
L = 16
NW = NC * NS
K = 80


def _sc_mesh():
    return plsc.VectorSubcoreMesh(core_axis_name="c", subcore_axis_name="s",
                                  num_cores=NC, num_subcores=NS)


@functools.lru_cache(maxsize=None)
def _make_deg_kernel(E, NP):
    EPW = E // NW
    KP = K
    NCH = EPW // KP
    RPT = NP // NS

    W = 8

    @functools.partial(
        pl.kernel,
        out_type=jax.ShapeDtypeStruct((NC, NP), jnp.float32),
        mesh=_sc_mesh(),
        compiler_params=pltpu.CompilerParams(needs_layout_passes=False),
        scratch_types=[
            pltpu.VMEM((KP,), jnp.float32),
            pltpu.VMEM((NCH, KP), jnp.int32),
            pltpu.VMEM((RPT,), jnp.float32),
            pltpu.VMEM_SHARED((NP,), jnp.float32),
            pltpu.SemaphoreType.DMA,
        ],
    )
    def deg_kernel(dst_hbm, out_hbm, ones_v, dstv, zbuf, acc, sem):
        cid = lax.axis_index("c")
        sid = lax.axis_index("s")
        w = cid * NS + sid

        pltpu.sync_copy(dst_hbm.at[w], dstv)

        def fill_zero(i, _):
            zbuf[pl.ds(i * L, L)] = jnp.zeros((L,), jnp.float32)
            return 0

        lax.fori_loop(0, RPT // L, fill_zero, 0)

        def fill_one(i, _):
            ones_v[pl.ds(i * L, L)] = jnp.ones((L,), jnp.float32)
            return 0

        lax.fori_loop(0, KP // L, fill_one, 0)

        pltpu.sync_copy(zbuf, acc.at[pl.ds(sid * RPT, RPT)])
        plsc.subcore_barrier()

        def drain_one():
            pltpu.make_async_copy(out_hbm.at[cid, pl.ds(0, KP)], ones_v,
                                  sem).wait()

        def step(j, _):
            pltpu.async_copy(ones_v, acc.at[dstv.at[j]], sem, add=True)

            @pl.when(j >= W)
            def _():
                drain_one()

            return 0

        lax.fori_loop(0, NCH, step, 0)

        def drain(j, _):
            drain_one()
            return 0

        lax.fori_loop(0, W, drain, 0)
        plsc.subcore_barrier()
        pltpu.sync_copy(acc.at[pl.ds(sid * RPT, RPT)],
                        out_hbm.at[cid, pl.ds(sid * RPT, RPT)])

    return deg_kernel


@functools.lru_cache(maxsize=None)
def _make_agg_kernel(E, NP, D):
    EPW = E // NW
    NCH = EPW // K
    RPT = NP // NS

    @functools.partial(
        pl.kernel,
        out_type=jax.ShapeDtypeStruct((NC, NP, D), jnp.float32),
        mesh=_sc_mesh(),
        compiler_params=pltpu.CompilerParams(needs_layout_passes=False),
        scratch_types=[
            pltpu.VMEM((EPW,), jnp.int32),
            pltpu.VMEM((NCH, K), jnp.int32),
            pltpu.VMEM((K, D), jnp.float32),
            pltpu.VMEM((K, D), jnp.float32),
            pltpu.VMEM_SHARED((NP, D), jnp.float32),
            pltpu.SemaphoreType.DMA,
            pltpu.SemaphoreType.DMA,
            pltpu.SemaphoreType.DMA,
            pltpu.SemaphoreType.DMA,
        ],
    )
    def agg_kernel(hs_hbm, src_hbm, dst_hbm, out_hbm,
                   srcv, dstv, rows0, rows1, acc, sem0, sem1, sems0, sems1):
        cid = lax.axis_index("c")
        sid = lax.axis_index("s")
        w = cid * NS + sid

        def fill_zero(i, _):
            def fill_row(k, _):
                rows0[i, pl.ds(k * L, L)] = jnp.zeros((L,), jnp.float32)
                return 0
            lax.fori_loop(0, D // L, fill_row, 0)
            return 0

        lax.fori_loop(0, K, fill_zero, 0)

        pltpu.sync_copy(src_hbm.at[pl.ds(w * EPW, EPW)], srcv)
        pltpu.sync_copy(dst_hbm.at[w], dstv)

        def zero_acc(t, _):
            pltpu.sync_copy(rows0, acc.at[pl.ds(sid * RPT + t * K, K)])
            return 0

        lax.fori_loop(0, RPT // K, zero_acc, 0)
        plsc.subcore_barrier()

        def wait_gather(j, rows, semx):
            pltpu.make_async_copy(hs_hbm.at[srcv.at[pl.ds(j * K, K)]],
                                  rows, semx).wait()

        def drain_scatter(semx):
            pltpu.make_async_copy(hs_hbm.at[pl.ds(0, K)], rows1, semx).wait()

        pltpu.async_copy(hs_hbm.at[srcv.at[pl.ds(0, K)]], rows0, sem0)
        pltpu.async_copy(hs_hbm.at[srcv.at[pl.ds(K, K)]], rows1, sem1)

        def pair(t, _):
            j0 = 2 * t
            wait_gather(j0, rows0, sem0)
            pltpu.async_copy(rows0, acc.at[dstv.at[j0]], sems0, add=True)
            wait_gather(j0 + 1, rows1, sem1)
            pltpu.async_copy(rows1, acc.at[dstv.at[j0 + 1]], sems1, add=True)
            drain_scatter(sems0)

            @pl.when(j0 + 2 < NCH)
            def _():
                pltpu.async_copy(hs_hbm.at[srcv.at[pl.ds((j0 + 2) * K, K)]],
                                 rows0, sem0)

            drain_scatter(sems1)

            @pl.when(j0 + 3 < NCH)
            def _():
                pltpu.async_copy(hs_hbm.at[srcv.at[pl.ds((j0 + 3) * K, K)]],
                                 rows1, sem1)

            return 0

        lax.fori_loop(0, NCH // 2, pair, 0)
        if NCH % 2 == 1:
            jl = NCH - 1
            wait_gather(jl, rows0, sem0)
            pltpu.async_copy(rows0, acc.at[dstv.at[jl]], sems0, add=True)
            drain_scatter(sems0)

        plsc.subcore_barrier()
        pltpu.sync_copy(acc.at[pl.ds(sid * RPT, RPT)],
                        out_hbm.at[cid, pl.ds(sid * RPT, RPT)])

    return agg_kernel


@functools.lru_cache(maxsize=None)
def _make_sagg_kernel(E, NP):
    EPW = E // NW
    KP = K
    NCH = EPW // KP
    RPT = NP // NS

    W = 8

    @functools.partial(
        pl.kernel,
        out_type=jax.ShapeDtypeStruct((NC, NP), jnp.float32),
        mesh=_sc_mesh(),
        compiler_params=pltpu.CompilerParams(needs_layout_passes=False),
        scratch_types=[
            pltpu.VMEM((NP,), jnp.float32),
            pltpu.VMEM((EPW,), jnp.int32),
            pltpu.VMEM((NCH, KP), jnp.int32),
            pltpu.VMEM((NCH, KP), jnp.float32),
            pltpu.VMEM((RPT,), jnp.float32),
            pltpu.VMEM_SHARED((NP,), jnp.float32),
            pltpu.SemaphoreType.DMA,
        ],
    )
    def sagg_kernel(zs_hbm, src_hbm, dst_hbm, out_hbm,
                    zsv, srcv, dstv, vals, zbuf, acc, sem):
        cid = lax.axis_index("c")
        sid = lax.axis_index("s")
        w = cid * NS + sid

        pltpu.sync_copy(zs_hbm, zsv)
        pltpu.sync_copy(src_hbm.at[pl.ds(w * EPW, EPW)], srcv)
        pltpu.sync_copy(dst_hbm.at[w], dstv)

        def fill_zero(i, _):
            zbuf[pl.ds(i * L, L)] = jnp.zeros((L,), jnp.float32)
            return 0

        lax.fori_loop(0, RPT // L, fill_zero, 0)
        pltpu.sync_copy(zbuf, acc.at[pl.ds(sid * RPT, RPT)])
        plsc.subcore_barrier()

        def drain_one():
            pltpu.make_async_copy(zs_hbm.at[pl.ds(0, KP)],
                                  zbuf.at[pl.ds(0, KP)], sem).wait()

        def step(j, _):
            def gath(i, _):
                idx16 = srcv[pl.ds(j * KP + i * L, L)]
                vals[j, pl.ds(i * L, L)] = plsc.load_gather(zsv, [idx16])
                return 0

            lax.fori_loop(0, KP // L, gath, 0)
            pltpu.async_copy(vals.at[j], acc.at[dstv.at[j]], sem, add=True)

            @pl.when(j >= W)
            def _():
                drain_one()

            return 0

        lax.fori_loop(0, NCH, step, 0)

        def drain(j, _):
            drain_one()
            return 0

        lax.fori_loop(0, W, drain, 0)
        plsc.subcore_barrier()
        pltpu.sync_copy(acc.at[pl.ds(sid * RPT, RPT)],
                        out_hbm.at[cid, pl.ds(sid * RPT, RPT)])

    return sagg_kernel


def _tc_hs_body(x_ref, w_ref, dp_ref, hs_ref, r_ref):
    NP, D = hs_ref.shape
    deg = dp_ref[0, :] + dp_ref[1, :] + 1.0
    rv = 1.0 / jnp.sqrt(deg)
    h = jnp.dot(x_ref[...].astype(jnp.bfloat16),
                w_ref[...].astype(jnp.bfloat16),
                preferred_element_type=jnp.float32)
    rb = lax.broadcast_in_dim(rv, (NP, D), (0,))
    hs_ref[...] = h * rb
    r_ref[...] = rv


def _tc_hs(xp, W1, degp):
    NP, D = xp.shape
    return pl.pallas_call(
        _tc_hs_body,
        out_shape=(jax.ShapeDtypeStruct((NP, D), jnp.float32),
                   jax.ShapeDtypeStruct((NP,), jnp.float32)),
    )(xp, W1, degp)


def _tc_zs_body(p_ref, hs_ref, r_ref, b1_ref, w2_ref, zs_ref):
    NP, D = hs_ref.shape
    s = p_ref[0] + p_ref[1] + hs_ref[...]
    rv = r_ref[...]
    rb = lax.broadcast_in_dim(rv, (NP, D), (0,))
    b1b = lax.broadcast_in_dim(b1_ref[...], (NP, D), (1,))
    h1 = jnp.maximum(s * rb + b1b, 0.0)
    h1q = h1.astype(jnp.bfloat16).astype(jnp.float32)
    w2q = w2_ref[...].astype(jnp.bfloat16).astype(jnp.float32)
    w2b = lax.broadcast_in_dim(w2q, (NP, D), (1,))
    z = jnp.sum(h1q * w2b, axis=1)
    zs_ref[...] = z * rv


def _tc_zs(P, hs, r, b1, w2):
    NP = hs.shape[0]
    return pl.pallas_call(
        _tc_zs_body,
        out_shape=jax.ShapeDtypeStruct((NP,), jnp.float32),
    )(P, hs, r, b1, w2)


def _tc_out_body(q_ref, zs_ref, r_ref, b2_ref, o_ref):
    s = q_ref[0] + q_ref[1] + zs_ref[...]
    o_ref[...] = s * r_ref[...] + b2_ref[...]


def _tc_out(Q, zs, r, b2):
    NP = zs.shape[0]
    return pl.pallas_call(
        _tc_out_body,
        out_shape=jax.ShapeDtypeStruct((NP,), jnp.float32),
    )(Q, zs, r, b2)


def kernel(x, edge_index, W1, b1, W2, b2):
    N, D = x.shape
    E = edge_index.shape[1]
    NP = ((N + 511) // 512) * 512

    src = edge_index[0]
    dst = edge_index[1]
    xp = jnp.pad(x, ((0, NP - N), (0, 0)))
    w2 = W2[:, 0]

    NCH = E // NW // K
    dst3 = dst.reshape(NW, NCH, K)

    degp = _make_deg_kernel(E, NP)(dst3)
    hs, r = _tc_hs(xp, W1, degp)
    P = _make_agg_kernel(E, NP, D)(hs, src, dst3)
    zs = _tc_zs(P, hs, r, b1, w2)
    Q = _make_sagg_kernel(E, NP)(zs, src, dst3)
    out = _tc_out(Q, zs, r, b2)
    return out[:N]


# speedup vs baseline: 1.1863x; 1.1863x over previous
import functools

import jax
import jax.numpy as jnp
from jax import lax
from jax.experimental import pallas as pl
from jax.experimental.pallas import tpu as pltpu
from jax.experimental.pallas import tpu_sc as plsc

NC = 2
NS = 16
L = 16
NW = NC * NS
K = 80


def _sc_mesh():
    return plsc.VectorSubcoreMesh(core_axis_name="c", subcore_axis_name="s",
                                  num_cores=NC, num_subcores=NS)


@functools.lru_cache(maxsize=None)
def _make_deg_kernel(E, NP):
    EPW = E // NW
    KP = K
    NCH = EPW // KP
    RPT = NP // NS

    W = 8

    @functools.partial(
        pl.kernel,
        out_type=jax.ShapeDtypeStruct((NC, NP), jnp.float32),
        mesh=_sc_mesh(),
        compiler_params=pltpu.CompilerParams(needs_layout_passes=False),
        scratch_types=[
            pltpu.VMEM((KP,), jnp.float32),
            pltpu.VMEM((NCH, KP), jnp.int32),
            pltpu.VMEM((RPT,), jnp.float32),
            pltpu.VMEM_SHARED((NP,), jnp.float32),
            pltpu.SemaphoreType.DMA,
        ],
    )
    def deg_kernel(dst_hbm, out_hbm, ones_v, dstv, zbuf, acc, sem):
        cid = lax.axis_index("c")
        sid = lax.axis_index("s")
        w = cid * NS + sid

        pltpu.sync_copy(dst_hbm.at[w], dstv)

        def fill_zero(i, _):
            zbuf[pl.ds(i * L, L)] = jnp.zeros((L,), jnp.float32)
            return 0

        lax.fori_loop(0, RPT // L, fill_zero, 0)

        def fill_one(i, _):
            ones_v[pl.ds(i * L, L)] = jnp.ones((L,), jnp.float32)
            return 0

        lax.fori_loop(0, KP // L, fill_one, 0)

        pltpu.sync_copy(zbuf, acc.at[pl.ds(sid * RPT, RPT)])
        plsc.subcore_barrier()

        def drain_one():
            pltpu.make_async_copy(out_hbm.at[cid, pl.ds(0, KP)], ones_v,
                                  sem).wait()

        def step(j, _):
            pltpu.async_copy(ones_v, acc.at[dstv.at[j]], sem, add=True)

            @pl.when(j >= W)
            def _():
                drain_one()

            return 0

        lax.fori_loop(0, NCH, step, 0)

        def drain(j, _):
            drain_one()
            return 0

        lax.fori_loop(0, W, drain, 0)
        plsc.subcore_barrier()
        pltpu.sync_copy(acc.at[pl.ds(sid * RPT, RPT)],
                        out_hbm.at[cid, pl.ds(sid * RPT, RPT)])

    return deg_kernel


@functools.lru_cache(maxsize=None)
def _make_agg_kernel(E, NP, D):
    EPW = E // NW
    NCH = EPW // K
    RPT = NP // NS

    @functools.partial(
        pl.kernel,
        out_type=jax.ShapeDtypeStruct((NC, NP, D), jnp.float32),
        mesh=_sc_mesh(),
        compiler_params=pltpu.CompilerParams(needs_layout_passes=False),
        scratch_types=[
            pltpu.VMEM((EPW,), jnp.int32),
            pltpu.VMEM((NCH, K), jnp.int32),
            pltpu.VMEM((K, D), jnp.float32),
            pltpu.VMEM((K, D), jnp.float32),
            pltpu.VMEM_SHARED((NP, D), jnp.float32),
            pltpu.SemaphoreType.DMA,
            pltpu.SemaphoreType.DMA,
        ],
    )
    def agg_kernel(hs_hbm, src_hbm, dst_hbm, out_hbm,
                   srcv, dstv, rows0, rows1, acc, sem0, sem1):
        cid = lax.axis_index("c")
        sid = lax.axis_index("s")
        w = cid * NS + sid

        def fill_zero(i, _):
            def fill_row(k, _):
                rows0[i, pl.ds(k * L, L)] = jnp.zeros((L,), jnp.float32)
                return 0
            lax.fori_loop(0, D // L, fill_row, 0)
            return 0

        lax.fori_loop(0, K, fill_zero, 0)

        pltpu.sync_copy(src_hbm.at[pl.ds(w * EPW, EPW)], srcv)
        pltpu.sync_copy(dst_hbm.at[w], dstv)

        def zero_acc(t, _):
            pltpu.sync_copy(rows0, acc.at[pl.ds(sid * RPT + t * K, K)])
            return 0

        lax.fori_loop(0, RPT // K, zero_acc, 0)
        plsc.subcore_barrier()

        pltpu.async_copy(hs_hbm.at[srcv.at[pl.ds(0, K)]], rows0, sem0)

        def pair(t, _):
            j0 = 2 * t
            pltpu.async_copy(hs_hbm.at[srcv.at[pl.ds((j0 + 1) * K, K)]], rows1, sem1)
            pltpu.make_async_copy(hs_hbm.at[srcv.at[pl.ds(j0 * K, K)]], rows0, sem0).wait()
            pltpu.sync_copy(rows0, acc.at[dstv.at[j0]], add=True)

            @pl.when(j0 + 2 < NCH)
            def _():
                pltpu.async_copy(hs_hbm.at[srcv.at[pl.ds((j0 + 2) * K, K)]], rows0, sem0)

            pltpu.make_async_copy(hs_hbm.at[srcv.at[pl.ds((j0 + 1) * K, K)]], rows1, sem1).wait()
            pltpu.sync_copy(rows1, acc.at[dstv.at[j0 + 1]], add=True)
            return 0

        lax.fori_loop(0, NCH // 2, pair, 0)
        if NCH % 2 == 1:
            jl = NCH - 1
            pltpu.make_async_copy(hs_hbm.at[srcv.at[pl.ds(jl * K, K)]], rows0, sem0).wait()
            pltpu.sync_copy(rows0, acc.at[dstv.at[jl]], add=True)

        plsc.subcore_barrier()
        pltpu.sync_copy(acc.at[pl.ds(sid * RPT, RPT)],
                        out_hbm.at[cid, pl.ds(sid * RPT, RPT)])

    return agg_kernel


@functools.lru_cache(maxsize=None)
def _make_sagg_kernel(E, NP):
    EPW = E // NW
    KP = K
    NCH = EPW // KP
    RPT = NP // NS

    W = 8

    @functools.partial(
        pl.kernel,
        out_type=jax.ShapeDtypeStruct((NC, NP), jnp.float32),
        mesh=_sc_mesh(),
        compiler_params=pltpu.CompilerParams(needs_layout_passes=False),
        scratch_types=[
            pltpu.VMEM((NP,), jnp.float32),
            pltpu.VMEM((EPW,), jnp.int32),
            pltpu.VMEM((NCH, KP), jnp.int32),
            pltpu.VMEM((NCH, KP), jnp.float32),
            pltpu.VMEM((RPT,), jnp.float32),
            pltpu.VMEM_SHARED((NP,), jnp.float32),
            pltpu.SemaphoreType.DMA,
        ],
    )
    def sagg_kernel(zs_hbm, src_hbm, dst_hbm, out_hbm,
                    zsv, srcv, dstv, vals, zbuf, acc, sem):
        cid = lax.axis_index("c")
        sid = lax.axis_index("s")
        w = cid * NS + sid

        pltpu.sync_copy(zs_hbm, zsv)
        pltpu.sync_copy(src_hbm.at[pl.ds(w * EPW, EPW)], srcv)
        pltpu.sync_copy(dst_hbm.at[w], dstv)

        def fill_zero(i, _):
            zbuf[pl.ds(i * L, L)] = jnp.zeros((L,), jnp.float32)
            return 0

        lax.fori_loop(0, RPT // L, fill_zero, 0)
        pltpu.sync_copy(zbuf, acc.at[pl.ds(sid * RPT, RPT)])
        plsc.subcore_barrier()

        def drain_one():
            pltpu.make_async_copy(zs_hbm.at[pl.ds(0, KP)],
                                  zbuf.at[pl.ds(0, KP)], sem).wait()

        def step(j, _):
            def gath(i, _):
                idx16 = srcv[pl.ds(j * KP + i * L, L)]
                vals[j, pl.ds(i * L, L)] = plsc.load_gather(zsv, [idx16])
                return 0

            lax.fori_loop(0, KP // L, gath, 0)
            pltpu.async_copy(vals.at[j], acc.at[dstv.at[j]], sem, add=True)

            @pl.when(j >= W)
            def _():
                drain_one()

            return 0

        lax.fori_loop(0, NCH, step, 0)

        def drain(j, _):
            drain_one()
            return 0

        lax.fori_loop(0, W, drain, 0)
        plsc.subcore_barrier()
        pltpu.sync_copy(acc.at[pl.ds(sid * RPT, RPT)],
                        out_hbm.at[cid, pl.ds(sid * RPT, RPT)])

    return sagg_kernel


def _tc_hs_body(x_ref, w_ref, dp_ref, hs_ref, r_ref):
    NP, D = hs_ref.shape
    deg = dp_ref[0, :] + dp_ref[1, :] + 1.0
    rv = 1.0 / jnp.sqrt(deg)
    h = jnp.dot(x_ref[...].astype(jnp.bfloat16),
                w_ref[...].astype(jnp.bfloat16),
                preferred_element_type=jnp.float32)
    rb = lax.broadcast_in_dim(rv, (NP, D), (0,))
    hs_ref[...] = h * rb
    r_ref[...] = rv


def _tc_hs(xp, W1, degp):
    NP, D = xp.shape
    return pl.pallas_call(
        _tc_hs_body,
        out_shape=(jax.ShapeDtypeStruct((NP, D), jnp.float32),
                   jax.ShapeDtypeStruct((NP,), jnp.float32)),
    )(xp, W1, degp)


def _tc_zs_body(p_ref, hs_ref, r_ref, b1_ref, w2_ref, zs_ref):
    NP, D = hs_ref.shape
    s = p_ref[0] + p_ref[1] + hs_ref[...]
    rv = r_ref[...]
    rb = lax.broadcast_in_dim(rv, (NP, D), (0,))
    b1b = lax.broadcast_in_dim(b1_ref[...], (NP, D), (1,))
    h1 = jnp.maximum(s * rb + b1b, 0.0)
    h1q = h1.astype(jnp.bfloat16).astype(jnp.float32)
    w2q = w2_ref[...].astype(jnp.bfloat16).astype(jnp.float32)
    w2b = lax.broadcast_in_dim(w2q, (NP, D), (1,))
    z = jnp.sum(h1q * w2b, axis=1)
    zs_ref[...] = z * rv


def _tc_zs(P, hs, r, b1, w2):
    NP = hs.shape[0]
    return pl.pallas_call(
        _tc_zs_body,
        out_shape=jax.ShapeDtypeStruct((NP,), jnp.float32),
    )(P, hs, r, b1, w2)


def _tc_out_body(q_ref, zs_ref, r_ref, b2_ref, o_ref):
    s = q_ref[0] + q_ref[1] + zs_ref[...]
    o_ref[...] = s * r_ref[...] + b2_ref[...]


def _tc_out(Q, zs, r, b2):
    NP = zs.shape[0]
    return pl.pallas_call(
        _tc_out_body,
        out_shape=jax.ShapeDtypeStruct((NP,), jnp.float32),
    )(Q, zs, r, b2)


def kernel(x, edge_index, W1, b1, W2, b2):
    N, D = x.shape
    E = edge_index.shape[1]
    NP = ((N + 511) // 512) * 512

    src = edge_index[0]
    dst = edge_index[1]
    xp = jnp.pad(x, ((0, NP - N), (0, 0)))
    w2 = W2[:, 0]

    NCH = E // NW // K
    dst3 = dst.reshape(NW, NCH, K)

    degp = _make_deg_kernel(E, NP)(dst3)
    hs, r = _tc_hs(xp, W1, degp)
    P = _make_agg_kernel(E, NP, D)(hs, src, dst3)
    zs = _tc_zs(P, hs, r, b1, w2)
    Q = _make_sagg_kernel(E, NP)(zs, src, dst3)
    out = _tc_out(Q, zs, r, b2)
    return out[:N]


# speedup vs baseline: 1.1902x; 1.0033x over previous
import functools

import jax
import jax.numpy as jnp
from jax import lax
from jax.experimental import pallas as pl
from jax.experimental.pallas import tpu as pltpu
from jax.experimental.pallas import tpu_sc as plsc

NC = 2
NS = 16
L = 16
NW = NC * NS
K = 80


def _sc_mesh():
    return plsc.VectorSubcoreMesh(core_axis_name="c", subcore_axis_name="s",
                                  num_cores=NC, num_subcores=NS)


@functools.lru_cache(maxsize=None)
def _make_deg_kernel(E, NP):
    EPW = E // NW
    KP = K
    NCH = EPW // KP
    RPT = NP // NS

    W = 8

    @functools.partial(
        pl.kernel,
        out_type=jax.ShapeDtypeStruct((NC, NP), jnp.float32),
        mesh=_sc_mesh(),
        compiler_params=pltpu.CompilerParams(needs_layout_passes=False),
        scratch_types=[
            pltpu.VMEM((KP,), jnp.float32),
            pltpu.VMEM((NCH, KP), jnp.int32),
            pltpu.VMEM((RPT,), jnp.float32),
            pltpu.VMEM_SHARED((NP,), jnp.float32),
            pltpu.SemaphoreType.DMA,
        ],
    )
    def deg_kernel(dst_hbm, out_hbm, ones_v, dstv, zbuf, acc, sem):
        cid = lax.axis_index("c")
        sid = lax.axis_index("s")
        w = cid * NS + sid

        pltpu.sync_copy(dst_hbm.at[w], dstv)

        def fill_zero(i, _):
            zbuf[pl.ds(i * L, L)] = jnp.zeros((L,), jnp.float32)
            return 0

        lax.fori_loop(0, RPT // L, fill_zero, 0)

        def fill_one(i, _):
            ones_v[pl.ds(i * L, L)] = jnp.ones((L,), jnp.float32)
            return 0

        lax.fori_loop(0, KP // L, fill_one, 0)

        pltpu.sync_copy(zbuf, acc.at[pl.ds(sid * RPT, RPT)])
        plsc.subcore_barrier()

        def drain_one():
            pltpu.make_async_copy(out_hbm.at[cid, pl.ds(0, KP)], ones_v,
                                  sem).wait()

        def step(j, _):
            pltpu.async_copy(ones_v, acc.at[dstv.at[j]], sem, add=True)

            @pl.when(j >= W)
            def _():
                drain_one()

            return 0

        lax.fori_loop(0, NCH, step, 0)

        def drain(j, _):
            drain_one()
            return 0

        lax.fori_loop(0, W, drain, 0)
        plsc.subcore_barrier()
        pltpu.sync_copy(acc.at[pl.ds(sid * RPT, RPT)],
                        out_hbm.at[cid, pl.ds(sid * RPT, RPT)])

    return deg_kernel


@functools.lru_cache(maxsize=None)
def _make_agg_kernel(E, NP, D):
    EPW = E // NW
    NCH = EPW // K
    RPT = NP // NS

    @functools.partial(
        pl.kernel,
        out_type=jax.ShapeDtypeStruct((NC, NP, D), jnp.float32),
        mesh=_sc_mesh(),
        compiler_params=pltpu.CompilerParams(needs_layout_passes=False),
        scratch_types=[
            pltpu.VMEM((EPW,), jnp.int32),
            pltpu.VMEM((NCH, K), jnp.int32),
            pltpu.VMEM((K, D), jnp.float32),
            pltpu.VMEM((K, D), jnp.float32),
            pltpu.VMEM_SHARED((NP, D), jnp.float32),
            pltpu.SemaphoreType.DMA,
            pltpu.SemaphoreType.DMA,
            pltpu.SemaphoreType.DMA,
        ],
    )
    def agg_kernel(hs_hbm, src_hbm, dst_hbm, out_hbm,
                   srcv, dstv, rows0, rows1, acc, sem0, sem1, semz):
        cid = lax.axis_index("c")
        sid = lax.axis_index("s")
        w = cid * NS + sid

        pltpu.sync_copy(src_hbm.at[pl.ds(w * EPW, EPW)], srcv)
        pltpu.async_copy(hs_hbm.at[srcv.at[pl.ds(0, K)]], rows0, sem0)
        pltpu.sync_copy(dst_hbm.at[w], dstv)

        def fill_zero(i, _):
            def fill_row(k, _):
                rows1[i, pl.ds(k * L, L)] = jnp.zeros((L,), jnp.float32)
                return 0
            lax.fori_loop(0, D // L, fill_row, 0)
            return 0

        lax.fori_loop(0, K, fill_zero, 0)

        def zero_acc(t, _):
            pltpu.async_copy(rows1, acc.at[pl.ds(sid * RPT + t * K, K)], semz)
            return 0

        lax.fori_loop(0, RPT // K, zero_acc, 0)

        def zero_drain(t, _):
            pltpu.make_async_copy(hs_hbm.at[pl.ds(0, K)], rows1, semz).wait()
            return 0

        lax.fori_loop(0, RPT // K, zero_drain, 0)
        plsc.subcore_barrier()

        pltpu.async_copy(hs_hbm.at[srcv.at[pl.ds(K, K)]], rows1, sem1)

        def pair(t, _):
            j0 = 2 * t
            pltpu.make_async_copy(hs_hbm.at[srcv.at[pl.ds(j0 * K, K)]], rows0, sem0).wait()
            pltpu.sync_copy(rows0, acc.at[dstv.at[j0]], add=True)

            @pl.when(j0 + 2 < NCH)
            def _():
                pltpu.async_copy(hs_hbm.at[srcv.at[pl.ds((j0 + 2) * K, K)]], rows0, sem0)

            pltpu.make_async_copy(hs_hbm.at[srcv.at[pl.ds((j0 + 1) * K, K)]], rows1, sem1).wait()
            pltpu.sync_copy(rows1, acc.at[dstv.at[j0 + 1]], add=True)

            @pl.when(j0 + 3 < NCH)
            def _():
                pltpu.async_copy(hs_hbm.at[srcv.at[pl.ds((j0 + 3) * K, K)]], rows1, sem1)

            return 0

        lax.fori_loop(0, NCH // 2, pair, 0)
        if NCH % 2 == 1:
            jl = NCH - 1
            pltpu.make_async_copy(hs_hbm.at[srcv.at[pl.ds(jl * K, K)]], rows0, sem0).wait()
            pltpu.sync_copy(rows0, acc.at[dstv.at[jl]], add=True)

        plsc.subcore_barrier()
        pltpu.sync_copy(acc.at[pl.ds(sid * RPT, RPT)],
                        out_hbm.at[cid, pl.ds(sid * RPT, RPT)])

    return agg_kernel


@functools.lru_cache(maxsize=None)
def _make_sagg_kernel(E, NP):
    EPW = E // NW
    KP = K
    NCH = EPW // KP
    RPT = NP // NS

    W = 8

    @functools.partial(
        pl.kernel,
        out_type=jax.ShapeDtypeStruct((NC, NP), jnp.float32),
        mesh=_sc_mesh(),
        compiler_params=pltpu.CompilerParams(needs_layout_passes=False),
        scratch_types=[
            pltpu.VMEM((NP,), jnp.float32),
            pltpu.VMEM((EPW,), jnp.int32),
            pltpu.VMEM((NCH, KP), jnp.int32),
            pltpu.VMEM((NCH, KP), jnp.float32),
            pltpu.VMEM((RPT,), jnp.float32),
            pltpu.VMEM_SHARED((NP,), jnp.float32),
            pltpu.SemaphoreType.DMA,
        ],
    )
    def sagg_kernel(zs_hbm, src_hbm, dst_hbm, out_hbm,
                    zsv, srcv, dstv, vals, zbuf, acc, sem):
        cid = lax.axis_index("c")
        sid = lax.axis_index("s")
        w = cid * NS + sid

        pltpu.sync_copy(zs_hbm, zsv)
        pltpu.sync_copy(src_hbm.at[pl.ds(w * EPW, EPW)], srcv)
        pltpu.sync_copy(dst_hbm.at[w], dstv)

        def fill_zero(i, _):
            zbuf[pl.ds(i * L, L)] = jnp.zeros((L,), jnp.float32)
            return 0

        lax.fori_loop(0, RPT // L, fill_zero, 0)
        pltpu.sync_copy(zbuf, acc.at[pl.ds(sid * RPT, RPT)])
        plsc.subcore_barrier()

        def drain_one():
            pltpu.make_async_copy(zs_hbm.at[pl.ds(0, KP)],
                                  zbuf.at[pl.ds(0, KP)], sem).wait()

        def step(j, _):
            def gath(i, _):
                idx16 = srcv[pl.ds(j * KP + i * L, L)]
                vals[j, pl.ds(i * L, L)] = plsc.load_gather(zsv, [idx16])
                return 0

            lax.fori_loop(0, KP // L, gath, 0)
            pltpu.async_copy(vals.at[j], acc.at[dstv.at[j]], sem, add=True)

            @pl.when(j >= W)
            def _():
                drain_one()

            return 0

        lax.fori_loop(0, NCH, step, 0)

        def drain(j, _):
            drain_one()
            return 0

        lax.fori_loop(0, W, drain, 0)
        plsc.subcore_barrier()
        pltpu.sync_copy(acc.at[pl.ds(sid * RPT, RPT)],
                        out_hbm.at[cid, pl.ds(sid * RPT, RPT)])

    return sagg_kernel


def _tc_hs_body(x_ref, w_ref, dp_ref, hs_ref, r_ref):
    NP, D = hs_ref.shape
    deg = dp_ref[0, :] + dp_ref[1, :] + 1.0
    rv = 1.0 / jnp.sqrt(deg)
    h = jnp.dot(x_ref[...].astype(jnp.bfloat16),
                w_ref[...].astype(jnp.bfloat16),
                preferred_element_type=jnp.float32)
    rb = lax.broadcast_in_dim(rv, (NP, D), (0,))
    hs_ref[...] = h * rb
    r_ref[...] = rv


def _tc_hs(xp, W1, degp):
    NP, D = xp.shape
    return pl.pallas_call(
        _tc_hs_body,
        out_shape=(jax.ShapeDtypeStruct((NP, D), jnp.float32),
                   jax.ShapeDtypeStruct((NP,), jnp.float32)),
    )(xp, W1, degp)


def _tc_zs_body(p_ref, hs_ref, r_ref, b1_ref, w2_ref, zs_ref):
    NP, D = hs_ref.shape
    s = p_ref[0] + p_ref[1] + hs_ref[...]
    rv = r_ref[...]
    rb = lax.broadcast_in_dim(rv, (NP, D), (0,))
    b1b = lax.broadcast_in_dim(b1_ref[...], (NP, D), (1,))
    h1 = jnp.maximum(s * rb + b1b, 0.0)
    h1q = h1.astype(jnp.bfloat16).astype(jnp.float32)
    w2q = w2_ref[...].astype(jnp.bfloat16).astype(jnp.float32)
    w2b = lax.broadcast_in_dim(w2q, (NP, D), (1,))
    z = jnp.sum(h1q * w2b, axis=1)
    zs_ref[...] = z * rv


def _tc_zs(P, hs, r, b1, w2):
    NP = hs.shape[0]
    return pl.pallas_call(
        _tc_zs_body,
        out_shape=jax.ShapeDtypeStruct((NP,), jnp.float32),
    )(P, hs, r, b1, w2)


def _tc_out_body(q_ref, zs_ref, r_ref, b2_ref, o_ref):
    s = q_ref[0] + q_ref[1] + zs_ref[...]
    o_ref[...] = s * r_ref[...] + b2_ref[...]


def _tc_out(Q, zs, r, b2):
    NP = zs.shape[0]
    return pl.pallas_call(
        _tc_out_body,
        out_shape=jax.ShapeDtypeStruct((NP,), jnp.float32),
    )(Q, zs, r, b2)


def kernel(x, edge_index, W1, b1, W2, b2):
    N, D = x.shape
    E = edge_index.shape[1]
    NP = ((N + 511) // 512) * 512

    src = edge_index[0]
    dst = edge_index[1]
    xp = jnp.pad(x, ((0, NP - N), (0, 0)))
    w2 = W2[:, 0]

    NCH = E // NW // K
    dst3 = dst.reshape(NW, NCH, K)

    degp = _make_deg_kernel(E, NP)(dst3)
    hs, r = _tc_hs(xp, W1, degp)
    P = _make_agg_kernel(E, NP, D)(hs, src, dst3)
    zs = _tc_zs(P, hs, r, b1, w2)
    Q = _make_sagg_kernel(E, NP)(zs, src, dst3)
    out = _tc_out(Q, zs, r, b2)
    return out[:N]
